# Initial kernel scaffold; baseline (speedup 1.0000x reference)
import jax
import jax.numpy as jnp
from jax.experimental import pallas as pl


def kernel(nApT, nAmT, X):
    raise NotImplementedError("write your pallas kernel here")


# speedup vs baseline: 1.8420x; 1.8420x over previous
import jax
import jax.numpy as jnp
from jax.experimental import pallas as pl
from jax.experimental.pallas import tpu as pltpu

_NUM_DIFF_LAYERS = 10
_C = 0.15
_BM = 200


def _diffusion_step_kernel(ap_ref, am_ref, p_ref, m_ref, tx_ref,
                           newp_ref, newm_ref):
    ap = ap_ref[...]
    am = am_ref[...]
    p = p_ref[...]
    m = m_ref[...]
    newp_ref[...] = (
        jnp.dot(ap, p, preferred_element_type=jnp.float32)
        + jnp.dot(am, m, preferred_element_type=jnp.float32)
        + tx_ref[...]
    )
    newm_ref[...] = (
        jnp.dot(am, p, preferred_element_type=jnp.float32)
        + jnp.dot(ap, m, preferred_element_type=jnp.float32)
    )


def _diffusion_step(ap, am, p, m, tx, bm):
    n, d = p.shape
    return pl.pallas_call(
        _diffusion_step_kernel,
        grid=(n // bm,),
        in_specs=[
            pl.BlockSpec((bm, n), lambda i: (i, 0)),
            pl.BlockSpec((bm, n), lambda i: (i, 0)),
            pl.BlockSpec((n, d), lambda i: (0, 0)),
            pl.BlockSpec((n, d), lambda i: (0, 0)),
            pl.BlockSpec((bm, d), lambda i: (i, 0)),
        ],
        out_specs=[
            pl.BlockSpec((bm, d), lambda i: (i, 0)),
            pl.BlockSpec((bm, d), lambda i: (i, 0)),
        ],
        out_shape=[
            jax.ShapeDtypeStruct((n, d), jnp.float32),
            jax.ShapeDtypeStruct((n, d), jnp.float32),
        ],
    )(ap, am, p, m, tx)


def kernel(nApT, nAmT, X):
    p = X
    m = jax.random.uniform(jax.random.key(1), X.shape, dtype=jnp.float32,
                           minval=-1.0, maxval=1.0)
    tx = _C * X
    for _ in range(_NUM_DIFF_LAYERS):
        p, m = _diffusion_step(nApT, nAmT, p, m, tx, _BM)
    return (p, m)
